# Initial kernel scaffold; baseline (speedup 1.0000x reference)
import jax
import jax.numpy as jnp
from jax.experimental import pallas as pl


def kernel(x, edge_index, batch, num_atoms, W1, b1, W2, b2, W3, b3, W4, b4):
    raise NotImplementedError("write your pallas kernel here")


# speedup vs baseline: 8.0117x; 8.0117x over previous
import functools

import jax
import jax.numpy as jnp
from jax import lax
from jax.experimental import pallas as pl
from jax.experimental.pallas import tpu as pltpu
from jax.experimental.pallas import tpu_sc as plsc

_N = 10000
_E = 320000
_D = 128
_G = 256
_NP = 10112
_JUNK = 10008
_CH = 128
_NW = 32
_EP = 327680
_EPW = _EP // _NW
_NITER = _EPW // _CH

_ROWS_Z = _NP // 16

_mesh = plsc.VectorSubcoreMesh(core_axis_name="c", subcore_axis_name="s")


@functools.partial(
    pl.kernel,
    out_type=jax.ShapeDtypeStruct((2, _NP, _D), jnp.float32),
    mesh=_mesh,
    scratch_types=[
        pltpu.VMEM((_CH,), jnp.int32),
        pltpu.VMEM((_CH,), jnp.int32),
        pltpu.VMEM((_CH, _D), jnp.float32),
        pltpu.VMEM_SHARED((_NP, _D), jnp.float32),
        pltpu.SemaphoreType.DMA,
    ],
)
def _sc_scatter(p_hbm, src_hbm, dst_hbm, zeros_hbm, out_hbm,
                sidx, didx, rows, acc, sem):
    c = lax.axis_index("c")
    s = lax.axis_index("s")
    wid = s * 2 + c
    pltpu.sync_copy(zeros_hbm.at[pl.ds(s * _ROWS_Z, _ROWS_Z)],
                    acc.at[pl.ds(s * _ROWS_Z, _ROWS_Z)])
    plsc.subcore_barrier()

    ebase = wid * _EPW

    def body(i, carry):
        off = ebase + i * _CH
        pltpu.sync_copy(src_hbm.at[pl.ds(off, _CH)], sidx)
        pltpu.sync_copy(dst_hbm.at[pl.ds(off, _CH)], didx)
        pltpu.async_copy(p_hbm.at[sidx], rows, sem).wait()
        pltpu.sync_copy(rows, acc.at[didx], add=True)
        return carry

    lax.fori_loop(0, _NITER, body, 0)
    plsc.subcore_barrier()
    pltpu.sync_copy(acc.at[pl.ds(s * _ROWS_Z, _ROWS_Z)],
                    out_hbm.at[c].at[pl.ds(s * _ROWS_Z, _ROWS_Z)])


@functools.partial(
    pl.kernel,
    out_type=jax.ShapeDtypeStruct((2, _NP, 16), jnp.float32),
    mesh=_mesh,
    scratch_types=[
        pltpu.VMEM((_CH,), jnp.int32),
        pltpu.VMEM((_CH, 16), jnp.float32),
        pltpu.VMEM_SHARED((_NP, 16), jnp.float32),
    ],
)
def _sc_degree(dst_hbm, ones_hbm, zeros_hbm, out_hbm, didx, ones_v, acc):
    c = lax.axis_index("c")
    s = lax.axis_index("s")
    wid = s * 2 + c
    pltpu.sync_copy(ones_hbm, ones_v)
    pltpu.sync_copy(zeros_hbm.at[pl.ds(s * _ROWS_Z, _ROWS_Z)],
                    acc.at[pl.ds(s * _ROWS_Z, _ROWS_Z)])
    plsc.subcore_barrier()

    ebase = wid * _EPW

    def body(i, carry):
        off = ebase + i * _CH
        pltpu.sync_copy(dst_hbm.at[pl.ds(off, _CH)], didx)
        pltpu.sync_copy(ones_v, acc.at[didx], add=True)
        return carry

    lax.fori_loop(0, _NITER, body, 0)
    plsc.subcore_barrier()
    pltpu.sync_copy(acc.at[pl.ds(s * _ROWS_Z, _ROWS_Z)],
                    out_hbm.at[c].at[pl.ds(s * _ROWS_Z, _ROWS_Z)])


_BLK = 1000
_NBLK = _N // _BLK


def _mm1_body(x_ref, w_ref, d0_ref, d1_ref, o_ref):
    dinv = lax.rsqrt(d0_ref[...] + d1_ref[...] + 1.0)
    o_ref[...] = jnp.dot(x_ref[...], w_ref[...],
                         preferred_element_type=jnp.float32) * dinv


def _mm1(x, w, d0, d1):
    return pl.pallas_call(
        _mm1_body,
        grid=(_NBLK,),
        in_specs=[
            pl.BlockSpec((_BLK, _D), lambda i: (i, 0)),
            pl.BlockSpec((_D, _D), lambda i: (0, 0)),
            pl.BlockSpec((_BLK, 1), lambda i: (i, 0)),
            pl.BlockSpec((_BLK, 1), lambda i: (i, 0)),
        ],
        out_specs=pl.BlockSpec((_BLK, _D), lambda i: (i, 0)),
        out_shape=jax.ShapeDtypeStruct((_N, _D), jnp.float32),
    )(x, w, d0, d1)


def _mm2_body(s0_ref, s1_ref, p_ref, d0_ref, d1_ref, w_ref, b_ref, o_ref):
    dinv = lax.rsqrt(d0_ref[...] + d1_ref[...] + 1.0)
    h = dinv * (s0_ref[...] + s1_ref[...] + p_ref[...]) + b_ref[...]
    h = jnp.maximum(h, 0.0)
    o_ref[...] = jnp.dot(h, w_ref[...],
                         preferred_element_type=jnp.float32) * dinv


def _mm2(s0, s1, p, d0, d1, w, b):
    return pl.pallas_call(
        _mm2_body,
        grid=(_NBLK,),
        in_specs=[
            pl.BlockSpec((_BLK, _D), lambda i: (i, 0)),
            pl.BlockSpec((_BLK, _D), lambda i: (i, 0)),
            pl.BlockSpec((_BLK, _D), lambda i: (i, 0)),
            pl.BlockSpec((_BLK, 1), lambda i: (i, 0)),
            pl.BlockSpec((_BLK, 1), lambda i: (i, 0)),
            pl.BlockSpec((_D, _D), lambda i: (0, 0)),
            pl.BlockSpec((1, _D), lambda i: (0, 0)),
        ],
        out_specs=pl.BlockSpec((_BLK, _D), lambda i: (i, 0)),
        out_shape=jax.ShapeDtypeStruct((_N, _D), jnp.float32),
    )(s0, s1, p, d0, d1, w, b)


_T = 12


def _head_body(s0_ref, s1_ref, p_ref, d0_ref, d1_ref, b2_ref, bat_ref,
               na_ref, w3a_ref, w3b_ref, b3_ref, w4_ref, b4_ref, o_ref,
               sums_ref, cnts_ref):
    i = pl.program_id(0)

    @pl.when(i == 0)
    def _():
        sums_ref[...] = jnp.zeros_like(sums_ref)
        cnts_ref[...] = jnp.zeros_like(cnts_ref)

    dinv = lax.rsqrt(d0_ref[...] + d1_ref[...] + 1.0)
    h = dinv * (s0_ref[...] + s1_ref[...] + p_ref[...]) + b2_ref[...]
    h = jnp.maximum(h, 0.0)

    seg = lax.broadcasted_iota(jnp.int32, (_BLK, _G), 1)
    oh = (bat_ref[...] == seg).astype(jnp.float32)
    sums_ref[...] += lax.dot_general(
        oh, h, (((0,), (0,)), ((), ())), preferred_element_type=jnp.float32)
    cnt = jnp.sum(oh, axis=0)
    cnts_ref[...] += jnp.broadcast_to(cnt[:, None], (_G, _D))

    @pl.when(i == _NBLK - 1)
    def _():
        mean = sums_ref[...] / jnp.maximum(cnts_ref[:, :1], 1.0)
        z = (jnp.dot(mean, w3a_ref[...], preferred_element_type=jnp.float32)
             + na_ref[...] * w3b_ref[...] + b3_ref[...])
        z = jnp.maximum(z, 0.0)
        o_ref[...] = jnp.dot(z, w4_ref[...],
                             preferred_element_type=jnp.float32) + b4_ref[...]


def _head(s0, s1, p, d0, d1, b2, bat, na, w3a, w3b, b3, w4, b4):
    return pl.pallas_call(
        _head_body,
        grid=(_NBLK,),
        in_specs=[
            pl.BlockSpec((_BLK, _D), lambda i: (i, 0)),
            pl.BlockSpec((_BLK, _D), lambda i: (i, 0)),
            pl.BlockSpec((_BLK, _D), lambda i: (i, 0)),
            pl.BlockSpec((_BLK, 1), lambda i: (i, 0)),
            pl.BlockSpec((_BLK, 1), lambda i: (i, 0)),
            pl.BlockSpec((1, _D), lambda i: (0, 0)),
            pl.BlockSpec((_BLK, 1), lambda i: (i, 0)),
            pl.BlockSpec((_G, 1), lambda i: (0, 0)),
            pl.BlockSpec((_D, 32), lambda i: (0, 0)),
            pl.BlockSpec((1, 32), lambda i: (0, 0)),
            pl.BlockSpec((1, 32), lambda i: (0, 0)),
            pl.BlockSpec((32, _T), lambda i: (0, 0)),
            pl.BlockSpec((1, _T), lambda i: (0, 0)),
        ],
        out_specs=pl.BlockSpec((_G, _T), lambda i: (0, 0)),
        out_shape=jax.ShapeDtypeStruct((_G, _T), jnp.float32),
        scratch_shapes=[
            pltpu.VMEM((_G, _D), jnp.float32),
            pltpu.VMEM((_G, _D), jnp.float32),
        ],
    )(s0, s1, p, d0, d1, b2, bat, na, w3a, w3b, b3, w4, b4)


def kernel(x, edge_index, batch, num_atoms, W1, b1, W2, b2, W3, b3, W4, b4):
    src = edge_index[0]
    dst = edge_index[1]
    pad = _EP - _E
    srcp = jnp.concatenate([src, jnp.zeros((pad,), jnp.int32)])
    dstp = jnp.concatenate([dst, jnp.full((pad,), _JUNK, jnp.int32)])

    zeros_big = jnp.zeros((_NP, _D), jnp.float32)
    zeros16 = jnp.zeros((_NP, 16), jnp.float32)
    ones16 = jnp.ones((_CH, 16), jnp.float32)

    degp = _sc_degree(dstp, ones16, zeros16)
    d0 = degp[0, :_N, :1]
    d1 = degp[1, :_N, :1]

    p1 = _mm1(x, W1, d0, d1)
    S1 = _sc_scatter(p1, srcp, dstp, zeros_big)
    p2 = _mm2(S1[0, :_N], S1[1, :_N], p1, d0, d1, W2, b1.reshape(1, -1))
    S2 = _sc_scatter(p2, srcp, dstp, zeros_big)

    return _head(S2[0, :_N], S2[1, :_N], p2, d0, d1, b2.reshape(1, -1),
                 batch.reshape(-1, 1), num_atoms,
                 W3[:_D], W3[_D:], b3.reshape(1, -1), W4, b4.reshape(1, -1))
